# baseline (speedup 1.0000x reference)
import functools

import jax
import jax.numpy as jnp
from jax.experimental import pallas as pl
from jax.experimental.pallas import tpu as pltpu

_BM = 400
_BK = 2560


def _body(xf_ref, adj_ref, xi_ref, w_ref, out_ref, acc_ref, deg_ref, *, nk, n):
    k = pl.program_id(1)

    @pl.when(k == 0)
    def _init():
        acc_ref[...] = jnp.zeros_like(acc_ref)
        deg_ref[...] = jnp.zeros_like(deg_ref)

    a = adj_ref[...]
    tail = n - (nk - 1) * _BK
    if tail != _BK:
        col = jax.lax.broadcasted_iota(jnp.int32, a.shape, 1)
        a = jnp.where((k != nk - 1) | (col < tail), a, 0.0)
    xk = xf_ref[pl.ds(k * _BK, _BK), :]
    acc_ref[...] += jnp.dot(a.astype(jnp.bfloat16), xk.astype(jnp.bfloat16),
                            preferred_element_type=jnp.float32)
    deg_ref[...] += jnp.sum(a, axis=1, keepdims=True)

    @pl.when(k == nk - 1)
    def _finish():
        d_in = xi_ref.shape[1]
        w = w_ref[...]
        agg = acc_ref[...] / jnp.clip(deg_ref[...], 1e-6, None)
        out_ref[...] = (
            jnp.dot(xi_ref[...], w[:d_in], preferred_element_type=jnp.float32)
            + jnp.dot(agg, w[d_in:], preferred_element_type=jnp.float32)
        )


def kernel(x, adj, W):
    n, d_in = x.shape
    d_out = W.shape[1]
    nm = pl.cdiv(n, _BM)
    nk = pl.cdiv(n, _BK)
    xf = jnp.pad(x, ((0, nk * _BK - n), (0, 0)))

    return pl.pallas_call(
        functools.partial(_body, nk=nk, n=n),
        grid=(nm, nk),
        in_specs=[
            pl.BlockSpec((nk * _BK, d_in), lambda i, k: (0, 0)),
            pl.BlockSpec((_BM, _BK), lambda i, k: (i, k)),
            pl.BlockSpec((_BM, d_in), lambda i, k: (i, 0)),
            pl.BlockSpec((2 * d_in, d_out), lambda i, k: (0, 0)),
        ],
        out_specs=pl.BlockSpec((_BM, d_out), lambda i, k: (i, 0)),
        out_shape=jax.ShapeDtypeStruct((n, d_out), jnp.float32),
        scratch_shapes=[
            pltpu.VMEM((_BM, d_out), jnp.float32),
            pltpu.VMEM((_BM, 1), jnp.float32),
        ],
        compiler_params=pltpu.CompilerParams(
            dimension_semantics=("parallel", "arbitrary"),
            vmem_limit_bytes=64 * 1024 * 1024,
        ),
    )(xf, adj, x, W)


# speedup vs baseline: 1.2797x; 1.2797x over previous
import jax
import jax.numpy as jnp
from jax.experimental import pallas as pl
from jax.experimental.pallas import tpu as pltpu

_BM = 240
_H = _BM // 2


def _half(a, xf, xi, w, d_in):
    acc = jnp.dot(a.astype(jnp.bfloat16), xf, preferred_element_type=jnp.float32)
    deg = jnp.sum(a, axis=1, keepdims=True)
    agg = acc / jnp.clip(deg, 1e-6, None)
    return (jnp.dot(xi, w[:d_in], preferred_element_type=jnp.float32)
            + jnp.dot(agg, w[d_in:], preferred_element_type=jnp.float32))


def _body(xf_ref, adj0_ref, adj1_ref, xi_ref, w_ref, out_ref):
    d_in = xi_ref.shape[1]
    xf = xf_ref[...].astype(jnp.bfloat16)
    w = w_ref[...]
    out_ref[: _H, :] = _half(adj0_ref[...], xf, xi_ref[: _H, :], w, d_in)
    out_ref[_H:, :] = _half(adj1_ref[...], xf, xi_ref[_H:, :], w, d_in)


def kernel(x, adj, W):
    n, d_in = x.shape
    d_out = W.shape[1]
    nm = pl.cdiv(n, _BM)

    return pl.pallas_call(
        _body,
        grid=(nm,),
        in_specs=[
            pl.BlockSpec((n, d_in), lambda i: (0, 0)),
            pl.BlockSpec((_H, n), lambda i: (2 * i, 0)),
            pl.BlockSpec((_H, n), lambda i: (2 * i + 1, 0)),
            pl.BlockSpec((_BM, d_in), lambda i: (i, 0)),
            pl.BlockSpec((2 * d_in, d_out), lambda i: (0, 0)),
        ],
        out_specs=pl.BlockSpec((_BM, d_out), lambda i: (i, 0)),
        out_shape=jax.ShapeDtypeStruct((n, d_out), jnp.float32),
        compiler_params=pltpu.CompilerParams(
            dimension_semantics=("parallel",),
            vmem_limit_bytes=64 * 1024 * 1024,
        ),
    )(x, adj, adj, x, W)
